# baseline (speedup 1.0000x reference)
import jax
import jax.numpy as jnp
from jax import lax
from jax.experimental import pallas as pl
from jax.experimental.pallas import tpu as pltpu

_VOCAB = 1000000
_EMB = 64
_CTX = 200

_NSEM = 16
_BV = 65536
_NB = (_VOCAB + _BV - 1) // _BV


def _body(idx_ref, et_ref, wt_ref, b_ref, raw_ref, lp_ref,
          bow_s, m_ref, s_ref, lse_ref, buf, sems):
  p = pl.program_id(0)
  i = pl.program_id(1)

  @pl.when(jnp.logical_and(p == 0, i == 0))
  def _():
    def _copy(j):
      base = pl.multiple_of((idx_ref[j] // 128) * 128, 128)
      return pltpu.make_async_copy(
          et_ref.at[:, pl.ds(base, 128)],
          buf.at[j],
          sems.at[j % _NSEM],
      )

    def _issue(j, carry):
      _copy(j).start()
      return carry

    def _drain(j, carry):
      _copy(j).wait()
      return carry

    lax.fori_loop(0, _CTX, _issue, 0)
    lax.fori_loop(0, _CTX, _drain, 0)

    lanes = lax.broadcasted_iota(jnp.int32, (_EMB, 128), 1)

    def _acc(j, acc128):
      lane = idx_ref[j] % 128
      return acc128 + jnp.where(lanes == lane, buf[j], 0.0)

    acc128 = lax.fori_loop(0, _CTX, _acc,
                           jnp.zeros((_EMB, 128), jnp.float32))
    bow_s[...] = jnp.sum(acc128, axis=1, keepdims=True)

  @pl.when(p == 0)
  def _():
    out = lax.dot_general(bow_s[...], wt_ref[...], (((0,), (0,)), ((), ())),
                          preferred_element_type=jnp.float32)
    out = out + b_ref[...][None, :]
    lp_ref[...] = out
    lane = lax.broadcasted_iota(jnp.int32, (1, _BV), 1)
    outm = jnp.where(lane < _VOCAB - i * _BV, out, -jnp.inf)
    bm = jnp.max(outm, keepdims=True)

    @pl.when(i == 0)
    def _():
      m_ref[...] = bm
      s_ref[...] = jnp.sum(jnp.exp(outm - bm), keepdims=True)

    @pl.when(i > 0)
    def _():
      m_old = m_ref[...]
      m_new = jnp.maximum(m_old, bm)
      s_ref[...] = (s_ref[...] * jnp.exp(m_old - m_new)
                    + jnp.sum(jnp.exp(outm - m_new), keepdims=True))
      m_ref[...] = m_new

    @pl.when(i == _NB - 1)
    def _():
      lse_ref[...] = m_ref[...] + jnp.log(s_ref[...])

  @pl.when(p == 1)
  def _():
    lp_ref[...] = raw_ref[...] - lse_ref[...]


def kernel(input, emb_table, W, b):
  idx = input.astype(jnp.int32)
  et = emb_table.T
  wt = W.T
  raw = jnp.zeros((1, _VOCAB), jnp.float32)
  last = _NB - 1
  return pl.pallas_call(
      _body,
      grid=(2, _NB),
      in_specs=[
          pl.BlockSpec(memory_space=pltpu.SMEM),
          pl.BlockSpec(memory_space=pl.ANY),
          pl.BlockSpec((_EMB, _BV),
                       lambda p, i: (0, jnp.where(p == 0, i, last))),
          pl.BlockSpec((_BV,), lambda p, i: (jnp.where(p == 0, i, last),)),
          pl.BlockSpec((1, _BV), lambda p, i: (0, jnp.where(p == 1, i, last))),
      ],
      out_specs=pl.BlockSpec((1, _BV), lambda p, i: (0, i)),
      out_shape=jax.ShapeDtypeStruct((1, _VOCAB), jnp.float32),
      scratch_shapes=[
          pltpu.VMEM((_EMB, 1), jnp.float32),
          pltpu.VMEM((1, 1), jnp.float32),
          pltpu.VMEM((1, 1), jnp.float32),
          pltpu.VMEM((1, 1), jnp.float32),
          pltpu.VMEM((_CTX, _EMB, 128), jnp.float32),
          pltpu.SemaphoreType.DMA((_NSEM,)),
      ],
      input_output_aliases={4: 0},
  )(idx, et, wt, b, raw)


# speedup vs baseline: 1.0125x; 1.0125x over previous
import jax
import jax.numpy as jnp
from jax import lax
from jax.experimental import pallas as pl
from jax.experimental.pallas import tpu as pltpu

_VOCAB = 1000000
_EMB = 64
_CTX = 200

_NSEM = 16
_BV = 65536
_NB = (_VOCAB + _BV - 1) // _BV


def _body(idx_ref, et_ref, wt_ref, b_ref, raw_ref, lp_ref,
          bow_s, m_ref, s_ref, lse_ref, buf, sems):
  p = pl.program_id(0)
  i = pl.program_id(1)

  @pl.when(jnp.logical_and(p == 0, i == 0))
  def _():
    def _copy(j):
      base = pl.multiple_of((idx_ref[j] // 128) * 128, 128)
      return pltpu.make_async_copy(
          et_ref.at[:, pl.ds(base, 128)],
          buf.at[j],
          sems.at[j % _NSEM],
      )

    def _issue(j, carry):
      _copy(j).start()
      return carry

    def _drain(j, carry):
      _copy(j).wait()
      return carry

    lax.fori_loop(0, _CTX, _issue, 0)
    lax.fori_loop(0, _CTX, _drain, 0)

    lanes = lax.broadcasted_iota(jnp.int32, (_EMB, 128), 1)

    def _acc(j, acc128):
      lane = idx_ref[j] % 128
      return acc128 + jnp.where(lanes == lane, buf[j], 0.0)

    acc128 = lax.fori_loop(0, _CTX, _acc,
                           jnp.zeros((_EMB, 128), jnp.float32))
    bow_s[...] = jnp.sum(acc128, axis=1, keepdims=True)

  @pl.when(p == 0)
  def _():
    out = lax.dot_general(bow_s[...], wt_ref[...], (((0,), (0,)), ((), ())),
                          preferred_element_type=jnp.float32)
    out = out + b_ref[...][None, :]
    lp_ref[...] = out
    lane = lax.broadcasted_iota(jnp.int32, (1, _BV), 1)
    outm = jnp.where(lane < _VOCAB - i * _BV, out, -jnp.inf)
    bm = jnp.max(outm, keepdims=True)

    @pl.when(i == 0)
    def _():
      m_ref[...] = bm
      s_ref[...] = jnp.sum(jnp.exp(outm - bm), keepdims=True)

    @pl.when(i > 0)
    def _():
      m_old = m_ref[...]
      m_new = jnp.maximum(m_old, bm)
      s_ref[...] = (s_ref[...] * jnp.exp(m_old - m_new)
                    + jnp.sum(jnp.exp(outm - m_new), keepdims=True))
      m_ref[...] = m_new

    @pl.when(i == _NB - 1)
    def _():
      lse_ref[...] = m_ref[...] + jnp.log(s_ref[...])

  @pl.when(p == 1)
  def _():
    lp_ref[...] = raw_ref[...] - lse_ref[...]


def _alloc_body(o_ref):
  pass


def kernel(input, emb_table, W, b):
  idx = input.astype(jnp.int32)
  et = emb_table.T
  wt = W.T
  raw = pl.pallas_call(
      _alloc_body,
      out_specs=pl.BlockSpec(memory_space=pl.ANY),
      out_shape=jax.ShapeDtypeStruct((1, _VOCAB), jnp.float32),
  )()
  last = _NB - 1
  return pl.pallas_call(
      _body,
      grid=(2, _NB),
      in_specs=[
          pl.BlockSpec(memory_space=pltpu.SMEM),
          pl.BlockSpec(memory_space=pl.ANY),
          pl.BlockSpec((_EMB, _BV),
                       lambda p, i: (0, jnp.where(p == 0, i, last))),
          pl.BlockSpec((_BV,), lambda p, i: (jnp.where(p == 0, i, last),)),
          pl.BlockSpec((1, _BV), lambda p, i: (0, jnp.where(p == 1, i, last))),
      ],
      out_specs=pl.BlockSpec((1, _BV), lambda p, i: (0, i)),
      out_shape=jax.ShapeDtypeStruct((1, _VOCAB), jnp.float32),
      scratch_shapes=[
          pltpu.VMEM((_EMB, 1), jnp.float32),
          pltpu.VMEM((1, 1), jnp.float32),
          pltpu.VMEM((1, 1), jnp.float32),
          pltpu.VMEM((1, 1), jnp.float32),
          pltpu.VMEM((_CTX, _EMB, 128), jnp.float32),
          pltpu.SemaphoreType.DMA((_NSEM,)),
      ],
      input_output_aliases={4: 0},
  )(idx, et, wt, b, raw)


# speedup vs baseline: 1.0477x; 1.0348x over previous
import jax
import jax.numpy as jnp
from jax import lax
from jax.experimental import pallas as pl
from jax.experimental.pallas import tpu as pltpu

_VOCAB = 1000000
_EMB = 64
_CTX = 200

_NSEM = 16
_BV = 65536
_NB = (_VOCAB + _BV - 1) // _BV
_BF = 131072
_NF = (_VOCAB + _BF - 1) // _BF


def _gather_body(idx_ref, et_ref, bow_ref, buf, sems):
  def _copy(j):
    base = pl.multiple_of((idx_ref[j] // 128) * 128, 128)
    return pltpu.make_async_copy(
        et_ref.at[:, pl.ds(base, 128)],
        buf.at[j],
        sems.at[j % _NSEM],
    )

  def _issue(j, carry):
    _copy(j).start()
    return carry

  def _drain(j, carry):
    _copy(j).wait()
    return carry

  lax.fori_loop(0, _CTX, _issue, 0)
  lax.fori_loop(0, _CTX, _drain, 0)

  lanes = lax.broadcasted_iota(jnp.int32, (_EMB, 128), 1)

  def _acc(j, acc128):
    lane = idx_ref[j] % 128
    return acc128 + jnp.where(lanes == lane, buf[j], 0.0)

  acc128 = lax.fori_loop(0, _CTX, _acc,
                         jnp.zeros((_EMB, 128), jnp.float32))
  bow_ref[...] = jnp.sum(acc128, axis=1, keepdims=True)


def _gather_pool(idx, et):
  return pl.pallas_call(
      _gather_body,
      in_specs=[
          pl.BlockSpec(memory_space=pltpu.SMEM),
          pl.BlockSpec(memory_space=pl.ANY),
      ],
      out_specs=pl.BlockSpec(memory_space=pltpu.VMEM),
      out_shape=jax.ShapeDtypeStruct((_EMB, 1), jnp.float32),
      scratch_shapes=[
          pltpu.VMEM((_CTX, _EMB, 128), jnp.float32),
          pltpu.SemaphoreType.DMA((_NSEM,)),
      ],
  )(idx, et)


def _matvec_body(bow_ref, wt_ref, b_ref, out_ref, lse_ref, m_ref, s_ref):
  i = pl.program_id(0)
  out = lax.dot_general(bow_ref[...], wt_ref[...], (((0,), (0,)), ((), ())),
                        preferred_element_type=jnp.float32)
  out = out + b_ref[...][None, :]
  out_ref[...] = out
  lane = lax.broadcasted_iota(jnp.int32, (1, _BV), 1)
  outm = jnp.where(lane < _VOCAB - i * _BV, out, -jnp.inf)
  bm = jnp.max(outm, keepdims=True)

  @pl.when(i == 0)
  def _():
    m_ref[...] = bm
    s_ref[...] = jnp.sum(jnp.exp(outm - bm), keepdims=True)

  @pl.when(i > 0)
  def _():
    m_old = m_ref[...]
    m_new = jnp.maximum(m_old, bm)
    s_ref[...] = (s_ref[...] * jnp.exp(m_old - m_new)
                  + jnp.sum(jnp.exp(outm - m_new), keepdims=True))
    m_ref[...] = m_new

  @pl.when(i == _NB - 1)
  def _():
    lse_ref[...] = m_ref[...] + jnp.log(s_ref[...])


def _matvec(bow, wt, b1):
  return pl.pallas_call(
      _matvec_body,
      grid=(_NB,),
      in_specs=[
          pl.BlockSpec((_EMB, 1), lambda i: (0, 0)),
          pl.BlockSpec((_EMB, _BV), lambda i: (0, i)),
          pl.BlockSpec((_BV,), lambda i: (i,)),
      ],
      out_specs=[
          pl.BlockSpec((1, _BV), lambda i: (0, i)),
          pl.BlockSpec((1, 1), lambda i: (0, 0)),
      ],
      out_shape=[
          jax.ShapeDtypeStruct((1, _VOCAB), jnp.float32),
          jax.ShapeDtypeStruct((1, 1), jnp.float32),
      ],
      scratch_shapes=[
          pltpu.VMEM((1, 1), jnp.float32),
          pltpu.VMEM((1, 1), jnp.float32),
      ],
  )(bow, wt, b1)


def _finalize_body(out_raw_ref, lse_ref, lp_ref):
  lp_ref[...] = out_raw_ref[...] - lse_ref[...]


def _finalize(out_raw, lse):
  return pl.pallas_call(
      _finalize_body,
      grid=(_NF,),
      in_specs=[
          pl.BlockSpec((1, _BF), lambda i: (0, i)),
          pl.BlockSpec((1, 1), lambda i: (0, 0)),
      ],
      out_specs=pl.BlockSpec((1, _BF), lambda i: (0, i)),
      out_shape=jax.ShapeDtypeStruct((1, _VOCAB), jnp.float32),
  )(out_raw, lse)


def kernel(input, emb_table, W, b):
  idx = input.astype(jnp.int32)
  et = emb_table.T
  wt = W.T
  bow = _gather_pool(idx, et)
  out_raw, lse = _matvec(bow, wt, b)
  return _finalize(out_raw, lse)
